# baseline (speedup 1.0000x reference)
import jax
import jax.numpy as jnp
from jax import lax
from jax.experimental import pallas as pl
from jax.experimental.pallas import tpu as pltpu
from jax.experimental.pallas import tpu_sc as plsc

B = 16384
D = 16
NC = 2
NS = 16
L = 16
NW = NC * NS
BPW = B // NW
CHUNK = 31
NCHUNK = -(-BPW // CHUNK)
PAD = 560


def _cmf_body(uid_hbm, iid_hbm, utab_hbm, itab_hbm, out_hbm,
              uid_v, iid_v, ubuf_v, ibuf_v, out_v, sem_u, sem_i):
    wid = lax.axis_index("s") * NC + lax.axis_index("c")
    base = wid * BPW

    lane = lax.iota(jnp.int32, L)
    zeros = jnp.zeros((L,), jnp.int32)
    for off in range(BPW, PAD, L):
        uid_v[pl.ds(off, L)] = zeros
        iid_v[pl.ds(off, L)] = zeros
    pltpu.sync_copy(uid_hbm.at[pl.ds(base, BPW)], uid_v.at[pl.ds(0, BPW)])
    pltpu.sync_copy(iid_hbm.at[pl.ds(base, BPW)], iid_v.at[pl.ds(0, BPW)])

    row2 = jnp.minimum(lane, CHUNK - 1 - L)

    def chunk_body(c, carry):
        idx0 = c * CHUNK
        uv0 = plsc.load_gather(uid_v, [idx0 + lane])
        iv0 = plsc.load_gather(iid_v, [idx0 + lane])
        uv1 = plsc.load_gather(uid_v, [idx0 + L + row2])
        iv1 = plsc.load_gather(iid_v, [idx0 + L + row2])
        cu0 = jnp.right_shift(uv0, 7) * 128
        ci0 = jnp.right_shift(iv0, 7) * 128
        cu1 = jnp.right_shift(uv1, 7) * 128
        ci1 = jnp.right_shift(iv1, 7) * 128
        copies = []
        for j in range(CHUNK):
            cu, ci, jl = (cu0, ci0, j) if j < L else (cu1, ci1, j - L)
            cuj = pl.multiple_of(jnp.sum(jnp.where(lane == jl, cu, 0)), 128)
            cij = pl.multiple_of(jnp.sum(jnp.where(lane == jl, ci, 0)), 128)
            copies.append(pltpu.async_copy(
                utab_hbm.at[:, pl.ds(cuj, 128)], ubuf_v.at[j], sem_u))
            copies.append(pltpu.async_copy(
                itab_hbm.at[:, pl.ds(cij, 128)], ibuf_v.at[j], sem_i))
        for cp in copies:
            cp.wait()

        for half, (uv, iv) in enumerate(((uv0, iv0), (uv1, iv1))):
            lu = jnp.bitwise_and(uv, 127)
            li = jnp.bitwise_and(iv, 127)
            r = lane if half == 0 else L + row2
            acc = jnp.zeros((L,), jnp.float32)
            for d in range(D):
                dsplat = jnp.full((L,), d, jnp.int32)
                u = plsc.load_gather(ubuf_v, [r, dsplat, lu])
                it = plsc.load_gather(ibuf_v, [r, dsplat, li])
                acc = acc + u * it
            res = 1.0 / (1.0 + jnp.exp(-acc))
            o = idx0 + lane if half == 0 else idx0 + L + row2
            plsc.store_scatter(out_v, [o], res)
        return carry

    lax.fori_loop(0, NCHUNK, chunk_body, 0)
    pltpu.sync_copy(out_v.at[pl.ds(0, BPW)], out_hbm.at[pl.ds(base, BPW)])


def kernel(user_ids, item_ids, source_user, source_item):
    mesh = plsc.VectorSubcoreMesh(
        core_axis_name="c", subcore_axis_name="s",
        num_cores=NC, num_subcores=NS)
    k = pl.kernel(
        _cmf_body,
        out_type=jax.ShapeDtypeStruct((B,), jnp.float32),
        mesh=mesh,
        compiler_params=pltpu.CompilerParams(
            needs_layout_passes=False, use_tc_tiling_on_sc=True),
        scratch_types=[
            pltpu.VMEM((PAD,), jnp.int32),
            pltpu.VMEM((PAD,), jnp.int32),
            pltpu.VMEM((CHUNK, D, 128), jnp.float32),
            pltpu.VMEM((CHUNK, D, 128), jnp.float32),
            pltpu.VMEM((PAD,), jnp.float32),
            pltpu.SemaphoreType.DMA,
            pltpu.SemaphoreType.DMA,
        ],
    )
    return k(user_ids.astype(jnp.int32), item_ids.astype(jnp.int32),
             source_user.T, source_item.T)


# speedup vs baseline: 1.1332x; 1.1332x over previous
import jax
import jax.numpy as jnp
from jax import lax
from jax.experimental import pallas as pl
from jax.experimental.pallas import tpu as pltpu
from jax.experimental.pallas import tpu_sc as plsc

B = 16384
D = 16
NC = 2
NS = 16
L = 16
NW = NC * NS
BPW = B // NW
CHUNK = 16
NCHUNK = BPW // CHUNK


def _cmf_body(uid_hbm, iid_hbm, utab_hbm, itab_hbm, out_hbm,
              uid_v, iid_v, ubuf_v, ibuf_v, out_v, sem_u, sem_i):
    wid = lax.axis_index("s") * NC + lax.axis_index("c")
    base = wid * BPW

    pltpu.sync_copy(uid_hbm.at[pl.ds(base, BPW)], uid_v)
    pltpu.sync_copy(iid_hbm.at[pl.ds(base, BPW)], iid_v)

    lane = lax.iota(jnp.int32, L)

    def chunk_body(b, carry):
        uvec = uid_v[pl.ds(b * CHUNK, CHUNK)]
        ivec = iid_v[pl.ds(b * CHUNK, CHUNK)]
        cu = jnp.right_shift(uvec, 7) * 128
        ci = jnp.right_shift(ivec, 7) * 128
        copies = []
        for j in range(CHUNK):
            cuj = pl.multiple_of(jnp.sum(jnp.where(lane == j, cu, 0)), 128)
            cij = pl.multiple_of(jnp.sum(jnp.where(lane == j, ci, 0)), 128)
            copies.append(pltpu.async_copy(
                utab_hbm.at[:, pl.ds(cuj, 128)], ubuf_v.at[j], sem_u))
            copies.append(pltpu.async_copy(
                itab_hbm.at[:, pl.ds(cij, 128)], ibuf_v.at[j], sem_i))
        for cp in copies:
            cp.wait()

        lu = jnp.bitwise_and(uvec, 127)
        li = jnp.bitwise_and(ivec, 127)
        acc = jnp.zeros((L,), jnp.float32)
        for d in range(D):
            dsplat = jnp.full((L,), d, jnp.int32)
            u = plsc.load_gather(ubuf_v, [lane, dsplat, lu])
            it = plsc.load_gather(ibuf_v, [lane, dsplat, li])
            acc = acc + u * it
        out_v[pl.ds(b * CHUNK, CHUNK)] = 1.0 / (1.0 + jnp.exp(-acc))
        return carry

    lax.fori_loop(0, NCHUNK, chunk_body, 0)
    pltpu.sync_copy(out_v, out_hbm.at[pl.ds(base, BPW)])


def kernel(user_ids, item_ids, source_user, source_item):
    mesh = plsc.VectorSubcoreMesh(
        core_axis_name="c", subcore_axis_name="s",
        num_cores=NC, num_subcores=NS)
    k = pl.kernel(
        _cmf_body,
        out_type=jax.ShapeDtypeStruct((B,), jnp.float32),
        mesh=mesh,
        compiler_params=pltpu.CompilerParams(
            needs_layout_passes=False, use_tc_tiling_on_sc=True),
        scratch_types=[
            pltpu.VMEM((BPW,), jnp.int32),
            pltpu.VMEM((BPW,), jnp.int32),
            pltpu.VMEM((CHUNK, D, 128), jnp.float32),
            pltpu.VMEM((CHUNK, D, 128), jnp.float32),
            pltpu.VMEM((BPW,), jnp.float32),
            pltpu.SemaphoreType.DMA,
            pltpu.SemaphoreType.DMA,
        ],
    )
    return k(user_ids.astype(jnp.int32), item_ids.astype(jnp.int32),
             source_user.T, source_item.T)
